# baseline (speedup 1.0000x reference)
import functools

import jax
import jax.numpy as jnp
from jax.experimental import pallas as pl
from jax.experimental.pallas import tpu as pltpu

_H = 24


def _edge_msg(e, hsrc, epw, epb, egw, egb, *, tile=2000, interpret=False):
    E, ED = e.shape

    def body(e_ref, h_ref, w_ref, b_ref, gw_ref, gb_ref, out_ref):
        eb = e_ref[...]
        proj = jnp.maximum(eb @ w_ref[...] + b_ref[...], 0.0)
        gate = jax.nn.sigmoid(eb @ gw_ref[...] + gb_ref[...])
        w = proj * gate
        h = h_ref[...]
        acc = h[:, 0:1] * w[:, 0:_H]
        for i in range(1, _H):
            acc = acc + h[:, i : i + 1] * w[:, i * _H : (i + 1) * _H]
        out_ref[...] = acc

    return pl.pallas_call(
        body,
        grid=(E // tile,),
        in_specs=[
            pl.BlockSpec((tile, ED), lambda i: (i, 0)),
            pl.BlockSpec((tile, _H), lambda i: (i, 0)),
            pl.BlockSpec((ED, ED), lambda i: (0, 0)),
            pl.BlockSpec((1, ED), lambda i: (0, 0)),
            pl.BlockSpec((ED, 1), lambda i: (0, 0)),
            pl.BlockSpec((1, 1), lambda i: (0, 0)),
        ],
        out_specs=pl.BlockSpec((tile, _H), lambda i: (i, 0)),
        out_shape=jax.ShapeDtypeStruct((E, _H), jnp.float32),
        interpret=interpret,
    )(e, hsrc, epw, epb.reshape(1, ED), egw, egb.reshape(1, 1))


def _interaction(sf, vf, slen_t, vlen, *, rt=200, interpret=False):
    N1, h = sf.shape
    N2 = vf.shape[0]
    I = N1 // rt

    def body(sf_ref, vf_ref, sl_ref, vl_ref, imap_ref, sp_ref, vp_ref, acc_ref):
        i = pl.program_id(0)
        sfb = sf_ref[...]
        vfb = vf_ref[...]
        t = jnp.tanh(jax.lax.dot_general(sfb, vfb, (((1,), (1,)), ((), ()))))
        t = t * sl_ref[...] * vl_ref[...]
        imap_ref[...] = t
        sp_ref[...] = t @ vfb
        vpc = jax.lax.dot_general(t, sfb, (((0,), (0,)), ((), ())))

        @pl.when(i == 0)
        def _():
            acc_ref[...] = vpc

        @pl.when(i > 0)
        def _():
            acc_ref[...] = acc_ref[...] + vpc

        @pl.when(i == I - 1)
        def _():
            vp_ref[...] = acc_ref[...]

    return pl.pallas_call(
        body,
        grid=(I,),
        in_specs=[
            pl.BlockSpec((rt, h), lambda i: (i, 0)),
            pl.BlockSpec((N2, h), lambda i: (0, 0)),
            pl.BlockSpec((rt, 1), lambda i: (i, 0)),
            pl.BlockSpec((1, N2), lambda i: (0, 0)),
        ],
        out_specs=[
            pl.BlockSpec((rt, N2), lambda i: (i, 0)),
            pl.BlockSpec((rt, h), lambda i: (i, 0)),
            pl.BlockSpec((N2, h), lambda i: (0, 0)),
        ],
        out_shape=[
            jax.ShapeDtypeStruct((N1, N2), jnp.float32),
            jax.ShapeDtypeStruct((N1, h), jnp.float32),
            jax.ShapeDtypeStruct((N2, h), jnp.float32),
        ],
        scratch_shapes=[pltpu.VMEM((N2, h), jnp.float32)],
        interpret=interpret,
    )(sf, vf, slen_t, vlen)


def _seg_sum(vals, idx, n):
    return jax.ops.segment_sum(vals, idx, num_segments=n)


def _rwse(src, dst, n, k=16):
    ones = jnp.ones((src.shape[0],), jnp.float32)
    deg = _seg_sum(ones, dst, n)
    deg = jnp.where(deg == 0, 1.0, deg)
    cols = [deg]
    prev = deg
    for _ in range(k - 1):
        prev = _seg_sum(prev[src], dst, n) / deg
        cols.append(prev)
    return jnp.stack(cols, axis=1)


def _set2set(feat, p, n_iters=2):
    d = feat.shape[1]
    hh = jnp.zeros((d,), jnp.float32)
    cc = jnp.zeros((d,), jnp.float32)
    q_star = jnp.zeros((2 * d,), jnp.float32)
    for _ in range(n_iters):
        gates = q_star @ p["w_ih"].T + p["b_ih"] + hh @ p["w_hh"].T + p["b_hh"]
        i, f, g, o = jnp.split(gates, 4)
        cc = jax.nn.sigmoid(f) * cc + jax.nn.sigmoid(i) * jnp.tanh(g)
        hh = jax.nn.sigmoid(o) * jnp.tanh(cc)
        e = feat @ hh
        alpha = jax.nn.softmax(e)
        readout = (feat * alpha[:, None]).sum(axis=0)
        q_star = jnp.concatenate([hh, readout])
    mean_feat = feat.mean(axis=0)
    return jnp.concatenate([q_star, mean_feat])[None, :]


def _gather_side(x, src, dst, e_feat, n, p):
    rwse = _rwse(src, dst, n)
    deg = rwse[:, 0]
    nf = jnp.concatenate([x, rwse], axis=1)
    out = jax.nn.relu(nf @ p["lin0_w"] + p["lin0_b"])
    if e_feat is not None:
        hsrc = out[src]
        msg = _edge_msg(e_feat, hsrc, p["ep_w"], p["ep_b"], p["eg_w"], p["eg_b"])
        agg = _seg_sum(msg, dst, n) / deg[:, None]
        m = jax.nn.relu(agg + out + p["conv_b"])
    else:
        m = jax.nn.relu(p["conv_b"] + out)
    out = jnp.concatenate([m, out], axis=1) @ p["msg_w"] + p["msg_b"]
    grp = _seg_sum(out[src], dst, n) / deg[:, None]
    out = jnp.concatenate([out, grp], axis=1) @ p["sub_w"] + p["sub_b"]
    return out + nf


def kernel(solute_x, solute_edge_index, solute_e, solvent_x, solvent_edge_index,
           solute_len, solvent_len, params):
    s_src, s_dst = solute_edge_index[0], solute_edge_index[1]
    v_src, v_dst = solvent_edge_index[0], solvent_edge_index[1]
    n1 = solute_x.shape[0]
    n2 = solvent_x.shape[0]

    sf = _gather_side(solute_x, s_src, s_dst, solute_e, n1, params["solute"])
    vf = _gather_side(solvent_x, v_src, v_dst, None, n2, params["solvent"])

    imap, sp, vp = _interaction(sf, vf, solute_len.T, solvent_len)

    sf2 = jnp.concatenate([sf, sp], axis=1)
    vf2 = jnp.concatenate([vf, vp], axis=1)
    ps = _set2set(sf2, params["s2s_solute"])
    pv = _set2set(vf2, params["s2s_solvent"])
    final = jnp.concatenate([ps, pv], axis=1)
    h1 = jax.nn.relu(final @ params["fc1_w"] + params["fc1_b"])
    h2 = jax.nn.relu(h1 @ params["fc2_w"] + params["fc2_b"])
    main = h2 @ params["fc3_w"] + params["fc3_b"]
    aux = (jax.nn.relu(final @ params["aux1_w"] + params["aux1_b"])
           @ params["aux2_w"] + params["aux2_b"])
    return main, aux, imap


# speedup vs baseline: 14.1331x; 14.1331x over previous
import functools

import jax
import jax.numpy as jnp
from jax import lax
from jax.experimental import pallas as pl
from jax.experimental.pallas import tpu as pltpu
from jax.experimental.pallas import tpu_sc as plsc

_H = 24
_F = 32
_NC = 2
_NS = 16
_SC_MESH = dict(core_axis_name="c", subcore_axis_name="s")


def _edge_msg(e, hsrc, epw, epb, egw, egb, *, tile=2000, interpret=False):
    E, ED = e.shape

    def body(e_ref, h_ref, w_ref, b_ref, gw_ref, gb_ref, out_ref):
        eb = e_ref[...]
        proj = jnp.maximum(eb @ w_ref[...] + b_ref[...], 0.0)
        gate = jax.nn.sigmoid(eb @ gw_ref[...] + gb_ref[...])
        w = proj * gate
        h = h_ref[...]
        acc = h[:, 0:1] * w[:, 0:_H]
        for i in range(1, _H):
            acc = acc + h[:, i : i + 1] * w[:, i * _H : (i + 1) * _H]
        out_ref[...] = jnp.concatenate(
            [acc, jnp.zeros((acc.shape[0], _F - _H), jnp.float32)], axis=1)

    hw = hsrc.shape[1]
    return pl.pallas_call(
        body,
        grid=(E // tile,),
        in_specs=[
            pl.BlockSpec((tile, ED), lambda i: (i, 0)),
            pl.BlockSpec((tile, hw), lambda i: (i, 0)),
            pl.BlockSpec((ED, ED), lambda i: (0, 0)),
            pl.BlockSpec((1, ED), lambda i: (0, 0)),
            pl.BlockSpec((ED, 1), lambda i: (0, 0)),
            pl.BlockSpec((1, 1), lambda i: (0, 0)),
        ],
        out_specs=pl.BlockSpec((tile, _F), lambda i: (i, 0)),
        out_shape=jax.ShapeDtypeStruct((E, _F), jnp.float32),
        interpret=interpret,
    )(e, hsrc, epw, epb.reshape(1, ED), egw, egb.reshape(1, 1))


def _interaction(sf, vf, slen_t, vlen, *, rt=200, interpret=False):
    N1, h = sf.shape
    N2 = vf.shape[0]
    I = N1 // rt

    def body(sf_ref, vf_ref, sl_ref, vl_ref, imap_ref, sp_ref, vp_ref, acc_ref):
        i = pl.program_id(0)
        sfb = sf_ref[...]
        vfb = vf_ref[...]
        t = jnp.tanh(jax.lax.dot_general(sfb, vfb, (((1,), (1,)), ((), ()))))
        t = t * sl_ref[...] * vl_ref[...]
        imap_ref[...] = t
        sp_ref[...] = t @ vfb
        vpc = jax.lax.dot_general(t, sfb, (((0,), (0,)), ((), ())))

        @pl.when(i == 0)
        def _():
            acc_ref[...] = vpc

        @pl.when(i > 0)
        def _():
            acc_ref[...] = acc_ref[...] + vpc

        @pl.when(i == I - 1)
        def _():
            vp_ref[...] = acc_ref[...]

    return pl.pallas_call(
        body,
        grid=(I,),
        in_specs=[
            pl.BlockSpec((rt, h), lambda i: (i, 0)),
            pl.BlockSpec((N2, h), lambda i: (0, 0)),
            pl.BlockSpec((rt, 1), lambda i: (i, 0)),
            pl.BlockSpec((1, N2), lambda i: (0, 0)),
        ],
        out_specs=[
            pl.BlockSpec((rt, N2), lambda i: (i, 0)),
            pl.BlockSpec((rt, h), lambda i: (i, 0)),
            pl.BlockSpec((N2, h), lambda i: (0, 0)),
        ],
        out_shape=[
            jax.ShapeDtypeStruct((N1, N2), jnp.float32),
            jax.ShapeDtypeStruct((N1, h), jnp.float32),
            jax.ShapeDtypeStruct((N2, h), jnp.float32),
        ],
        scratch_shapes=[pltpu.VMEM((N2, h), jnp.float32)],
        interpret=interpret,
    )(sf, vf, slen_t, vlen)


def _sc_rwse(s_src, s_dst, v_src, v_dst, n, k=16):
    e = s_src.shape[0]
    eps = e // _NS
    npad = ((n + _NS * 16 - 1) // (_NS * 16)) * (_NS * 16)
    nps = npad // _NS
    nv = nps // 16

    def graph(cid, src_hbm, dst_hbm, ones_hbm, zer_hbm, out_hbm,
              src_v, dst_v, vals_v, col_v, deg_v, zer_v, pbuf, acc, sem):
        sid = lax.axis_index("s")
        sl = pl.ds(sid * nps, nps)
        pltpu.sync_copy(src_hbm.at[pl.ds(sid * eps, eps)], src_v)
        pltpu.sync_copy(dst_hbm.at[pl.ds(sid * eps, eps)], dst_v)
        pltpu.sync_copy(ones_hbm, vals_v)
        pltpu.sync_copy(zer_hbm, zer_v)
        pltpu.sync_copy(zer_v, acc.at[sl])
        plsc.subcore_barrier()
        pltpu.sync_copy(vals_v, acc.at[dst_v], add=True)
        plsc.subcore_barrier()
        pltpu.sync_copy(acc.at[sl], col_v)

        @pl.loop(0, nv)
        def _(j):
            v = col_v[pl.ds(j * 16, 16)]
            v = jnp.where(v == 0.0, 1.0, v)
            col_v[pl.ds(j * 16, 16)] = v
            deg_v[pl.ds(j * 16, 16)] = v

        pltpu.sync_copy(col_v, pbuf.at[sl])
        pltpu.sync_copy(col_v, out_hbm.at[cid, 0, sl])
        plsc.subcore_barrier()
        for kk in range(1, k):
            pltpu.sync_copy(zer_v, acc.at[sl])
            pltpu.async_copy(pbuf.at[src_v], vals_v, sem).wait()
            plsc.subcore_barrier()
            pltpu.sync_copy(vals_v, acc.at[dst_v], add=True)
            plsc.subcore_barrier()
            pltpu.sync_copy(acc.at[sl], col_v)

            @pl.loop(0, nv)
            def _(j):
                v = col_v[pl.ds(j * 16, 16)] / deg_v[pl.ds(j * 16, 16)]
                col_v[pl.ds(j * 16, 16)] = v

            pltpu.sync_copy(col_v, pbuf.at[sl])
            pltpu.sync_copy(col_v, out_hbm.at[cid, kk, sl])
            plsc.subcore_barrier()

    @functools.partial(
        pl.kernel,
        out_type=jax.ShapeDtypeStruct((2, k, npad), jnp.float32),
        mesh=plsc.VectorSubcoreMesh(**_SC_MESH),
        scratch_types=[
            pltpu.VMEM((eps,), jnp.int32),
            pltpu.VMEM((eps,), jnp.int32),
            pltpu.VMEM((eps,), jnp.float32),
            pltpu.VMEM((nps,), jnp.float32),
            pltpu.VMEM((nps,), jnp.float32),
            pltpu.VMEM((nps,), jnp.float32),
            pltpu.VMEM_SHARED((npad,), jnp.float32),
            pltpu.VMEM_SHARED((npad,), jnp.float32),
            pltpu.SemaphoreType.DMA,
        ],
    )
    def run(ss_hbm, sd_hbm, vs_hbm, vd_hbm, ones_hbm, zer_hbm, out_hbm,
            src_v, dst_v, vals_v, col_v, deg_v, zer_v, pbuf, acc, sem):
        cid = lax.axis_index("c")

        @pl.when(cid == 0)
        def _():
            graph(0, ss_hbm, sd_hbm, ones_hbm, zer_hbm, out_hbm,
                  src_v, dst_v, vals_v, col_v, deg_v, zer_v, pbuf, acc, sem)

        @pl.when(cid == 1)
        def _():
            graph(1, vs_hbm, vd_hbm, ones_hbm, zer_hbm, out_hbm,
                  src_v, dst_v, vals_v, col_v, deg_v, zer_v, pbuf, acc, sem)

    ones = jnp.ones((eps,), jnp.float32)
    zer = jnp.zeros((nps,), jnp.float32)
    out = run(s_src, s_dst, v_src, v_dst, ones, zer)
    rwse_s = out[0, :, :n].T
    rwse_v = out[1, :, :n].T
    return rwse_s, rwse_v


def _sc_gather(table, idx, *, chunk=1000):
    n, f = table.shape
    e = idx.shape[0]
    epw = e // (_NC * _NS)
    nch = epw // chunk

    @functools.partial(
        pl.kernel,
        out_type=jax.ShapeDtypeStruct((e, f), jnp.float32),
        mesh=plsc.VectorSubcoreMesh(**_SC_MESH),
        compiler_params=pltpu.CompilerParams(use_tc_tiling_on_sc=False),
        scratch_types=[
            pltpu.VMEM((chunk,), jnp.int32),
            pltpu.VMEM((chunk, f), jnp.float32),
            pltpu.SemaphoreType.DMA,
        ],
    )
    def run(table_hbm, idx_hbm, out_hbm, idx_v, rows_v, sem):
        wid = lax.axis_index("s") * _NC + lax.axis_index("c")
        for ch in range(nch):
            base = wid * epw + ch * chunk
            pltpu.sync_copy(idx_hbm.at[pl.ds(base, chunk)], idx_v)
            pltpu.async_copy(table_hbm.at[idx_v], rows_v, sem).wait()
            pltpu.sync_copy(rows_v, out_hbm.at[pl.ds(base, chunk)])

    return run(table, idx)


def _sc_scatter_sum(rows, dst, n, src=None, *, chunk=1000):
    f = rows.shape[1]
    e = dst.shape[0]
    epc = e // _NC
    eps = epc // _NS
    nch = eps // chunk
    npad = ((n + _NS * 16 - 1) // (_NS * 16)) * (_NS * 16)
    nps = npad // _NS
    gather = src is not None

    @functools.partial(
        pl.kernel,
        out_type=jax.ShapeDtypeStruct((_NC, npad, f), jnp.float32),
        mesh=plsc.VectorSubcoreMesh(**_SC_MESH),
        compiler_params=pltpu.CompilerParams(use_tc_tiling_on_sc=False),
        scratch_types=[
            pltpu.VMEM((chunk,), jnp.int32),
            pltpu.VMEM((chunk,), jnp.int32),
            pltpu.VMEM((chunk, f), jnp.float32),
            pltpu.VMEM((nps, f), jnp.float32),
            pltpu.VMEM_SHARED((npad, f), jnp.float32),
            pltpu.SemaphoreType.DMA,
        ],
    )
    def run(rows_hbm, dst_hbm, src_hbm, zer_hbm, out_hbm,
            didx_v, sidx_v, rows_v, zer_v, acc, sem):
        cid = lax.axis_index("c")
        sid = lax.axis_index("s")
        sl = pl.ds(sid * nps, nps)
        pltpu.sync_copy(zer_hbm, zer_v)
        pltpu.sync_copy(zer_v, acc.at[sl])
        plsc.subcore_barrier()
        for ch in range(nch):
            base = cid * epc + sid * eps + ch * chunk
            pltpu.sync_copy(dst_hbm.at[pl.ds(base, chunk)], didx_v)
            if gather:
                pltpu.sync_copy(src_hbm.at[pl.ds(base, chunk)], sidx_v)
                pltpu.async_copy(rows_hbm.at[sidx_v], rows_v, sem).wait()
            else:
                pltpu.sync_copy(rows_hbm.at[pl.ds(base, chunk)], rows_v)
            pltpu.sync_copy(rows_v, acc.at[didx_v], add=True)
        plsc.subcore_barrier()
        pltpu.sync_copy(acc.at[sl], zer_v)
        pltpu.sync_copy(zer_v, out_hbm.at[cid, sl])

    zer = jnp.zeros((nps, f), jnp.float32)
    src_in = src if gather else dst
    out = run(rows, dst, src_in, zer)
    return out[0, :n] + out[1, :n]


def _seg_sum(vals, idx, n):
    return jax.ops.segment_sum(vals, idx, num_segments=n)


def _pad_f(x):
    return jnp.pad(x, ((0, 0), (0, _F - x.shape[1])))


def _set2set(feat, p, n_iters=2):
    d = feat.shape[1]
    hh = jnp.zeros((d,), jnp.float32)
    cc = jnp.zeros((d,), jnp.float32)
    q_star = jnp.zeros((2 * d,), jnp.float32)
    for _ in range(n_iters):
        gates = q_star @ p["w_ih"].T + p["b_ih"] + hh @ p["w_hh"].T + p["b_hh"]
        i, f, g, o = jnp.split(gates, 4)
        cc = jax.nn.sigmoid(f) * cc + jax.nn.sigmoid(i) * jnp.tanh(g)
        hh = jax.nn.sigmoid(o) * jnp.tanh(cc)
        e = feat @ hh
        alpha = jax.nn.softmax(e)
        readout = (feat * alpha[:, None]).sum(axis=0)
        q_star = jnp.concatenate([hh, readout])
    mean_feat = feat.mean(axis=0)
    return jnp.concatenate([q_star, mean_feat])[None, :]


def _gather_side(x, src, dst, e_feat, n, p, rwse):
    deg = rwse[:, 0]
    nf = jnp.concatenate([x, rwse], axis=1)
    out = jax.nn.relu(nf @ p["lin0_w"] + p["lin0_b"])
    if e_feat is not None:
        hsrc = _sc_gather(_pad_f(out), src)
        msg = _edge_msg(e_feat, hsrc, p["ep_w"], p["ep_b"], p["eg_w"], p["eg_b"])
        agg = _sc_scatter_sum(msg, dst, n)[:, :_H] / deg[:, None]
        m = jax.nn.relu(agg + out + p["conv_b"])
    else:
        m = jax.nn.relu(p["conv_b"] + out)
    out = jnp.concatenate([m, out], axis=1) @ p["msg_w"] + p["msg_b"]
    grp = _sc_scatter_sum(_pad_f(out), dst, n, src=src)[:, :_H] / deg[:, None]
    out = jnp.concatenate([out, grp], axis=1) @ p["sub_w"] + p["sub_b"]
    return out + nf


def kernel(solute_x, solute_edge_index, solute_e, solvent_x, solvent_edge_index,
           solute_len, solvent_len, params):
    s_src, s_dst = solute_edge_index[0], solute_edge_index[1]
    v_src, v_dst = solvent_edge_index[0], solvent_edge_index[1]
    n1 = solute_x.shape[0]
    n2 = solvent_x.shape[0]

    rwse_s, rwse_v = _sc_rwse(s_src, s_dst, v_src, v_dst, n1)
    sf = _gather_side(solute_x, s_src, s_dst, solute_e, n1, params["solute"],
                      rwse_s)
    vf = _gather_side(solvent_x, v_src, v_dst, None, n2, params["solvent"],
                      rwse_v)

    imap, sp, vp = _interaction(sf, vf, solute_len.T, solvent_len)

    sf2 = jnp.concatenate([sf, sp], axis=1)
    vf2 = jnp.concatenate([vf, vp], axis=1)
    ps = _set2set(sf2, params["s2s_solute"])
    pv = _set2set(vf2, params["s2s_solvent"])
    final = jnp.concatenate([ps, pv], axis=1)
    h1 = jax.nn.relu(final @ params["fc1_w"] + params["fc1_b"])
    h2 = jax.nn.relu(h1 @ params["fc2_w"] + params["fc2_b"])
    main = h2 @ params["fc3_w"] + params["fc3_b"]
    aux = (jax.nn.relu(final @ params["aux1_w"] + params["aux1_b"])
           @ params["aux2_w"] + params["aux2_b"])
    return main, aux, imap
